# Initial kernel scaffold; baseline (speedup 1.0000x reference)
import jax
import jax.numpy as jnp
from jax.experimental import pallas as pl


def kernel(node_features, edge_indices, edge_features, params):
    raise NotImplementedError("write your pallas kernel here")


# speedup vs baseline: 3.8004x; 3.8004x over previous
import jax
import jax.numpy as jnp
import numpy as np
from jax import lax
from jax.experimental import pallas as pl
from jax.experimental.pallas import tpu as pltpu

N, E, ND, ED, HD, L = 256, 4096, 256, 16, 256, 6
ECHUNK = 512
NCHUNKS = E // ECHUNK
BI = 8

_OFF_IDX = np.nonzero(~np.eye(N, dtype=bool).reshape(-1))[0]

F32 = jnp.float32
BF16 = jnp.bfloat16


def _dotb(a, b, dn=None):
    a = a.astype(BF16)
    b = b.astype(BF16)
    if dn is None:
        return lax.dot(a, b, preferred_element_type=F32)
    return lax.dot_general(a, b, dimension_numbers=dn,
                           preferred_element_type=F32)


def _split3(v):
    hi = v.astype(BF16)
    r1 = v - hi.astype(F32)
    mid = r1.astype(BF16)
    lo = (r1 - mid.astype(F32)).astype(BF16)
    return hi, mid, lo


def _gnn_kernel(
    x0_ref, ef_ref, src_ref, dst_ref,
    node_W_ref, node_b_ref, edge_W_ref, edge_b_ref,
    msg_W1_ref, msg_b1_ref, msg_W2_ref, msg_b2_ref,
    upd_W1_ref, upd_b1_ref, upd_W2_ref, upd_b2_ref,
    ln_g_ref, ln_b_ref,
    glob_W_ref, glob_b_ref,
    move_W1_ref, move_b1_ref, move_W2_ref, move_b2_ref,
    shoot_W1_ref, shoot_b1_ref, shoot_W2_ref, shoot_b2_ref,
    tgt_W1_ref, tgt_b1_ref, tgt_W2_ref, tgt_b2_ref,
    mv_ref, probs_ref, sv_ref, scores_ref, x_ref,
    oh_src_ref, oh_dst_ref, a_ref, b_ref,
):
    def build_oh(c, _):
        e0 = c * ECHUNK
        iota_n = lax.broadcasted_iota(jnp.int32, (ECHUNK, N), 1)
        s = src_ref[pl.ds(e0, ECHUNK), :]
        d = dst_ref[pl.ds(e0, ECHUNK), :]
        oh_src_ref[pl.ds(e0, ECHUNK), :] = (s == iota_n).astype(BF16)
        oh_dst_ref[pl.ds(e0, ECHUNK), :] = (d == iota_n).astype(BF16)
        return 0

    lax.fori_loop(0, NCHUNKS, build_oh, 0, unroll=1)

    x = x0_ref[...]

    for i in range(L):
        W1a = msg_W1_ref[i, :HD, :]
        W1b = msg_W1_ref[i, HD:, :]
        h = _dotb(x, node_W_ref[i]) + node_b_ref[i]
        hW1 = _dotb(h, W1a)
        hW1_3 = _split3(hW1)
        b1 = msg_b1_ref[i]
        b2 = msg_b2_ref[i]
        W2 = msg_W2_ref[i]

        def edge_chunk(c, agg):
            e0 = c * ECHUNK
            oh_s = oh_src_ref[pl.ds(e0, ECHUNK), :]
            oh_d = oh_dst_ref[pl.ds(e0, ECHUNK), :]
            e = _dotb(ef_ref[pl.ds(e0, ECHUNK), :], edge_W_ref[i]) + edge_b_ref[i]
            eW1 = _dotb(e, W1b)
            gath = (lax.dot(oh_s, hW1_3[0], preferred_element_type=F32)
                    + lax.dot(oh_s, hW1_3[1], preferred_element_type=F32)
                    + lax.dot(oh_s, hW1_3[2], preferred_element_type=F32))
            pre = gath + eW1 + b1
            s = jnp.maximum(pre, 0.0)
            m = _dotb(s, W2) + b2
            dn = (((0,), (0,)), ((), ()))
            for t in _split3(m):
                agg = agg + lax.dot_general(oh_d, t, dimension_numbers=dn,
                                            preferred_element_type=F32)
            return agg

        agg = lax.fori_loop(0, NCHUNKS, edge_chunk,
                            jnp.zeros((N, HD), F32), unroll=1)

        up = (_dotb(x, upd_W1_ref[i, :ND, :]) + _dotb(agg, upd_W1_ref[i, ND:, :])
              + upd_b1_ref[i])
        x2 = _dotb(jnp.maximum(up, 0.0), upd_W2_ref[i]) + upd_b2_ref[i]
        mu = jnp.mean(x2, axis=-1, keepdims=True)
        var = jnp.mean((x2 - mu) ** 2, axis=-1, keepdims=True)
        x2 = (x2 - mu) / jnp.sqrt(var + 1e-5) * ln_g_ref[i] + ln_b_ref[i]
        x = jnp.maximum(x2 + x, 0.0)

    x_ref[...] = x

    gvec = (_dotb(jnp.mean(x, axis=0, keepdims=True), glob_W_ref[...])
            + glob_b_ref[...])

    mv_pre = jnp.maximum(
        _dotb(x, move_W1_ref[:HD, :])
        + (_dotb(gvec, move_W1_ref[HD:, :]) + move_b1_ref[...]), 0.0)
    mv = _dotb(mv_pre, move_W2_ref[...]) + move_b2_ref[...]
    mv_ref[...] = mv
    mmax = jnp.max(mv, axis=-1, keepdims=True)
    ex = jnp.exp(mv - mmax)
    probs_ref[...] = ex / jnp.sum(ex, axis=-1, keepdims=True)

    sv_pre = jnp.maximum(
        _dotb(x, shoot_W1_ref[:HD, :])
        + (_dotb(gvec, shoot_W1_ref[HD:, :]) + shoot_b1_ref[...]), 0.0)
    sv = _dotb(sv_pre, shoot_W2_ref[...]) + shoot_b2_ref[...]
    sv_ref[...] = jnp.clip(sv, -10.0, 10.0)

    a_ref[...] = _dotb(x, tgt_W1_ref[:HD, :]) + (
        _dotb(gvec, tgt_W1_ref[2 * HD:, :]) + tgt_b1_ref[...])
    b_ref[...] = _dotb(x, tgt_W1_ref[HD:2 * HD, :])
    w2row = tgt_W2_ref[...].astype(BF16).astype(F32)
    tb2 = tgt_b2_ref[0, 0]

    def score_blk(c, _):
        i0 = c * BI
        a_blk = a_ref[pl.ds(i0, BI), :]
        hid = jnp.maximum(a_blk[:, None, :] + b_ref[...][None, :, :], 0.0)
        hid = hid.astype(BF16).astype(F32)
        sc = jnp.sum(hid * w2row[None, :, :], axis=-1)
        scores_ref[pl.ds(i0, BI), :] = jnp.clip(sc + tb2, -10.0, 10.0)
        return 0

    lax.fori_loop(0, N // BI, score_blk, 0, unroll=1)


@jax.jit
def kernel(node_features, edge_indices, edge_features, params):
    p = params
    src = edge_indices[0].reshape(E, 1).astype(jnp.int32)
    dst = edge_indices[1].reshape(E, 1).astype(jnp.int32)

    def b2d(name):
        return p[name].reshape(L, 1, -1)

    out_shapes = (
        jax.ShapeDtypeStruct((N, 4), F32),
        jax.ShapeDtypeStruct((N, 4), F32),
        jax.ShapeDtypeStruct((N, 1), F32),
        jax.ShapeDtypeStruct((N, N), F32),
        jax.ShapeDtypeStruct((N, HD), F32),
    )
    scratch = [
        pltpu.VMEM((E, N), BF16),
        pltpu.VMEM((E, N), BF16),
        pltpu.VMEM((N, HD), F32),
        pltpu.VMEM((N, HD), F32),
    ]

    mv, probs, sv, scores, x = pl.pallas_call(
        _gnn_kernel,
        out_shape=out_shapes,
        scratch_shapes=scratch,
    )(
        node_features, edge_features, src, dst,
        p["node_W"], b2d("node_b"), p["edge_W"], b2d("edge_b"),
        p["msg_W1"], b2d("msg_b1"), p["msg_W2"], b2d("msg_b2"),
        p["upd_W1"], b2d("upd_b1"), p["upd_W2"], b2d("upd_b2"),
        b2d("ln_g"), b2d("ln_b"),
        p["glob_W"], p["glob_b"].reshape(1, HD),
        p["move_W1"], p["move_b1"].reshape(1, HD),
        p["move_W2"], p["move_b2"].reshape(1, 4),
        p["shoot_W1"], p["shoot_b1"].reshape(1, HD),
        p["shoot_W2"], p["shoot_b2"].reshape(1, 1),
        p["tgt_W1"], p["tgt_b1"].reshape(1, HD),
        p["tgt_W2"].reshape(1, HD), p["tgt_b2"].reshape(1, 1),
    )

    ts = scores.reshape(-1)[_OFF_IDX]
    return (mv, probs, sv.reshape(N), ts, x)
